# baseline (speedup 1.0000x reference)
import functools

import jax
import jax.numpy as jnp
import numpy as np
from jax import lax
from jax.experimental import pallas as pl
from jax.experimental.pallas import tpu as pltpu
from jax.experimental.pallas import tpu_sc as plsc

N = 50000
R = 16
H1 = 16
E = N * 32
RN = R * N

L = 16
NC = 2
NS = 16
NW = NC * NS

GRP = 128
CG = 10
CE = CG * GRP
NGRP = E // GRP
NCHUNK = NGRP // CG

CS_PER_TILE = RN // NS
ROWS_PER_TILE = N // NS
ZF = 2000
ZR = 25


def _sc_body(toh, s2d, wf, parts,
             colsum_sh, out_sh,
             frbuf, to2d, vals1, rows,
             ones1, zrow, zflat, vsem, rsem, ssem):
    cid = lax.axis_index("c")
    sid = lax.axis_index("s")
    wid = sid * NC + cid

    ones16 = jnp.ones((L,), jnp.float32)
    zeros16 = jnp.zeros((L,), jnp.float32)
    for i in range(GRP // L):
        ones1[pl.ds(i * L, L)] = ones16
    for r in range(ZR):
        zrow[r, :] = zeros16

    @pl.loop(0, ZF // L)
    def _zfill(i):
        zflat[pl.ds(i * L, L)] = jnp.zeros((L,), jnp.float32)

    @pl.loop(0, CS_PER_TILE // ZF)
    def _zero_cs(k):
        pltpu.sync_copy(zflat, colsum_sh.at[pl.ds(sid * CS_PER_TILE + k * ZF, ZF)])

    @pl.loop(0, ROWS_PER_TILE // ZR)
    def _zero_out(k):
        pltpu.sync_copy(zrow, out_sh.at[pl.ds(sid * ROWS_PER_TILE + k * ZR, ZR), :])

    plsc.subcore_barrier()

    def _compute_to(chunk, with_fr):
        pltpu.sync_copy(toh.at[pl.ds(chunk * CG, CG), :], to2d)
        if with_fr:
            pltpu.sync_copy(s2d.at[pl.ds(chunk * CG, CG), :], frbuf)

    _nsA = jax.named_scope("phaseA_count"); _nsA.__enter__()

    @pl.loop(sid, NCHUNK, step=NS)
    def _count(chunk):
        _compute_to(chunk, with_fr=False)
        descs = [pltpu.async_copy(ones1, colsum_sh.at[to2d.at[j]], ssem, add=True)
                 for j in range(CG)]
        for d in descs:
            d.wait()

    _nsA.__exit__(None, None, None)
    plsc.subcore_barrier()

    _nsA2 = jax.named_scope("phaseA2_recip"); _nsA2.__enter__()
    @pl.loop(0, CS_PER_TILE // ZF)
    def _recip(k):
        sl = pl.ds(sid * CS_PER_TILE + k * ZF, ZF)
        pltpu.sync_copy(colsum_sh.at[sl], zflat)

        @pl.loop(0, ZF // L)
        def _inv(i):
            zflat[pl.ds(i * L, L)] = 1.0 / zflat[pl.ds(i * L, L)]

        pltpu.sync_copy(zflat, colsum_sh.at[sl])

    _nsA2.__exit__(None, None, None)
    plsc.subcore_barrier()

    _nsB = jax.named_scope("phaseB_main"); _nsB.__enter__()
    @pl.loop(wid, NCHUNK, step=NW)
    def _process(chunk):
        _compute_to(chunk, with_fr=True)
        vdescs = [pltpu.async_copy(colsum_sh.at[to2d.at[j]],
                                   vals1.at[pl.ds(j * GRP, GRP)], vsem)
                  for j in range(CG)]
        rdescs = [pltpu.async_copy(wf.at[to2d.at[j]],
                                   rows.at[pl.ds(j * GRP, GRP), :], rsem)
                  for j in range(CG)]
        for d in vdescs:
            d.wait()
        for d in rdescs:
            d.wait()

        @pl.loop(0, CE // L)
        def _scale(g):
            e0 = g * L
            v16 = vals1[pl.ds(e0, L)]
            for t in range(L):
                rows[e0 + t, :] = rows[e0 + t, :] * v16[t]

        sdescs = [pltpu.async_copy(rows.at[pl.ds(j * GRP, GRP), :],
                                   out_sh.at[frbuf.at[j]], ssem, add=True)
                  for j in range(CG)]
        for d in sdescs:
            d.wait()

    _nsB.__exit__(None, None, None)
    plsc.subcore_barrier()

    pltpu.sync_copy(out_sh.at[pl.ds(sid * ROWS_PER_TILE, ROWS_PER_TILE), :],
                    parts.at[cid, pl.ds(sid * ROWS_PER_TILE, ROWS_PER_TILE), :])


@functools.partial(jax.jit, static_argnames=())
def _sc_call(toh, s2d, wflat):
    mesh = plsc.VectorSubcoreMesh(core_axis_name="c", subcore_axis_name="s",
                                  num_cores=NC, num_subcores=NS)
    f = pl.kernel(
        _sc_body,
        out_type=jax.ShapeDtypeStruct((NC, N, H1), jnp.float32),
        mesh=mesh,
        scratch_types=[
            pltpu.VMEM_SHARED((RN,), jnp.float32),
            pltpu.VMEM_SHARED((N, H1), jnp.float32),
            pltpu.VMEM((CG, GRP), jnp.int32),
            pltpu.VMEM((CG, GRP), jnp.int32),
            pltpu.VMEM((CE,), jnp.float32),
            pltpu.VMEM((CE, H1), jnp.float32),
            pltpu.VMEM((GRP,), jnp.float32),
            pltpu.VMEM((ZR, H1), jnp.float32),
            pltpu.VMEM((ZF,), jnp.float32),
            pltpu.SemaphoreType.DMA,
            pltpu.SemaphoreType.DMA,
            pltpu.SemaphoreType.DMA,
        ],
        compiler_params=pltpu.CompilerParams(use_tc_tiling_on_sc=False),
    )
    return f(toh, s2d, wflat)


def _combine_body(parts_ref, bias_ref, out_ref):
    out_ref[...] = parts_ref[0] + parts_ref[1] + bias_ref[...]


BR = 2000


def _combine(parts, bias2):
    return pl.pallas_call(
        _combine_body,
        out_shape=jax.ShapeDtypeStruct((N, H1), jnp.float32),
        grid=(N // BR,),
        in_specs=[
            pl.BlockSpec((NC, BR, H1), lambda i: (0, i, 0)),
            pl.BlockSpec((1, H1), lambda i: (0, 0)),
        ],
        out_specs=pl.BlockSpec((BR, H1), lambda i: (i, 0)),
    )(parts, bias2)


_SELTO = np.zeros((3 * GRP, GRP), np.float32)
_SELS = np.zeros((3 * GRP, GRP), np.float32)
for _k in range(GRP):
    _SELTO[3 * _k + 1, _k] = float(N)
    _SELTO[3 * _k + 2, _k] = 1.0
    _SELS[3 * _k, _k] = 1.0
_SELTOJ = jnp.asarray(_SELTO)
_SELSJ = jnp.asarray(_SELS)


def kernel(triples, weights, bias):
    tri = triples.reshape(NGRP, 3 * GRP).astype(jnp.float32)
    toh = (lax.dot(tri, _SELTOJ, precision=lax.Precision.HIGHEST)
           + 0.5).astype(jnp.int32)
    s2d = (lax.dot(tri, _SELSJ, precision=lax.Precision.HIGHEST)
           + 0.5).astype(jnp.int32)
    wflat = weights.reshape(RN, H1)
    parts = _sc_call(toh, s2d, wflat)
    return _combine(parts, bias.reshape(1, H1))


# speedup vs baseline: 8.1661x; 8.1661x over previous
import functools

import jax
import jax.numpy as jnp
from jax import lax
from jax.experimental import pallas as pl
from jax.experimental.pallas import tpu as pltpu
from jax.experimental.pallas import tpu_sc as plsc

N = 50000
R = 16
H1 = 16
E = N * 32
RN = R * N

L = 16
NC = 2
NS = 16
NW = NC * NS

GRP = 128
CG = 10
CE = CG * GRP
NGRP = E // GRP
NCHUNK = NGRP // CG

CS_PER_TILE = RN // NS
ROWS_PER_TILE = N // NS
ZF = 2000
ZR = 25


def _sc_body(toh, s2d, wf, parts,
             colsum_sh, out_sh,
             frbuf, to2d, vals1, rows,
             ones1, zrow, zflat, zflat2, vsem, rsem, ssem, isem):
    cid = lax.axis_index("c")
    sid = lax.axis_index("s")
    wid = sid * NC + cid

    ones16 = jnp.ones((L,), jnp.float32)
    zeros16 = jnp.zeros((L,), jnp.float32)
    for i in range(GRP // L):
        ones1[pl.ds(i * L, L)] = ones16
    for r in range(ZR):
        zrow[r, :] = zeros16

    @pl.loop(0, ZF // L)
    def _zfill(i):
        zflat[pl.ds(i * L, L)] = jnp.zeros((L,), jnp.float32)

    @pl.loop(0, CS_PER_TILE // ZF)
    def _zero_cs(k):
        pltpu.sync_copy(zflat, colsum_sh.at[pl.ds(sid * CS_PER_TILE + k * ZF, ZF)])

    @pl.loop(0, ROWS_PER_TILE // ZR)
    def _zero_out(k):
        pltpu.sync_copy(zrow, out_sh.at[pl.ds(sid * ROWS_PER_TILE + k * ZR, ZR), :])

    plsc.subcore_barrier()

    _nsA = jax.named_scope("phaseA_count"); _nsA.__enter__()

    @pl.loop(sid, NCHUNK, step=2 * NS)
    def _count(c0):
        pltpu.sync_copy(toh.at[pl.ds(c0 * CG, CG), :], to2d)
        d0 = [pltpu.async_copy(ones1, colsum_sh.at[to2d.at[j]], ssem, add=True)
              for j in range(CG)]
        c1 = c0 + NS

        @pl.when(c1 < NCHUNK)
        def _second():
            pltpu.sync_copy(toh.at[pl.ds(c1 * CG, CG), :], frbuf)
            d1 = [pltpu.async_copy(ones1, colsum_sh.at[frbuf.at[j]], vsem,
                                   add=True)
                  for j in range(CG)]
            for d in d1:
                d.wait()

        for d in d0:
            d.wait()

    _nsA.__exit__(None, None, None)
    plsc.subcore_barrier()

    _nsA2 = jax.named_scope("phaseA2_recip"); _nsA2.__enter__()
    NSL = CS_PER_TILE // ZF

    @pl.loop(0, NSL - 1, step=2)
    def _recip(k):
        sl0 = pl.ds(sid * CS_PER_TILE + k * ZF, ZF)
        sl1 = pl.ds(sid * CS_PER_TILE + (k + 1) * ZF, ZF)
        l0 = pltpu.async_copy(colsum_sh.at[sl0], zflat, vsem)
        l1 = pltpu.async_copy(colsum_sh.at[sl1], zflat2, rsem)
        l0.wait()

        @pl.loop(0, ZF // L)
        def _inv0(i):
            zflat[pl.ds(i * L, L)] = 1.0 / zflat[pl.ds(i * L, L)]

        s0 = pltpu.async_copy(zflat, colsum_sh.at[sl0], vsem)
        l1.wait()

        @pl.loop(0, ZF // L)
        def _inv1(i):
            zflat2[pl.ds(i * L, L)] = 1.0 / zflat2[pl.ds(i * L, L)]

        s1 = pltpu.async_copy(zflat2, colsum_sh.at[sl1], rsem)
        s0.wait()
        s1.wait()

    if NSL % 2 == 1:
        slt = pl.ds(sid * CS_PER_TILE + (NSL - 1) * ZF, ZF)
        pltpu.sync_copy(colsum_sh.at[slt], zflat)

        @pl.loop(0, ZF // L)
        def _invt(i):
            zflat[pl.ds(i * L, L)] = 1.0 / zflat[pl.ds(i * L, L)]

        pltpu.sync_copy(zflat, colsum_sh.at[slt])

    _nsA2.__exit__(None, None, None)
    plsc.subcore_barrier()

    _nsB = jax.named_scope("phaseB_main"); _nsB.__enter__()

    @pl.loop(wid, NCHUNK, step=NW)
    def _process(chunk):
        i0 = pltpu.async_copy(toh.at[pl.ds(chunk * CG, CG), :], to2d, isem)
        i1 = pltpu.async_copy(s2d.at[pl.ds(chunk * CG, CG), :], frbuf, isem)
        i0.wait()
        i1.wait()
        vdescs = [pltpu.async_copy(colsum_sh.at[to2d.at[j]],
                                   vals1.at[pl.ds(j * GRP, GRP)], vsem)
                  for j in range(CG)]
        rdescs = [pltpu.async_copy(wf.at[to2d.at[j]],
                                   rows.at[pl.ds(j * GRP, GRP), :], rsem)
                  for j in range(CG)]
        sdescs = []
        for j in range(CG):
            vdescs[j].wait()
            rdescs[j].wait()

            @pl.loop(0, GRP // L)
            def _scale(g):
                e0 = j * GRP + g * L
                v16 = vals1[pl.ds(e0, L)]
                for t in range(L):
                    rows[e0 + t, :] = rows[e0 + t, :] * v16[t]

            sdescs.append(
                pltpu.async_copy(rows.at[pl.ds(j * GRP, GRP), :],
                                 out_sh.at[frbuf.at[j]], ssem, add=True))
        for d in sdescs:
            d.wait()

    _nsB.__exit__(None, None, None)
    plsc.subcore_barrier()

    pltpu.sync_copy(out_sh.at[pl.ds(sid * ROWS_PER_TILE, ROWS_PER_TILE), :],
                    parts.at[cid, pl.ds(sid * ROWS_PER_TILE, ROWS_PER_TILE), :])


@functools.partial(jax.jit, static_argnames=())
def _sc_call(toh, s2d, wflat):
    mesh = plsc.VectorSubcoreMesh(core_axis_name="c", subcore_axis_name="s",
                                  num_cores=NC, num_subcores=NS)
    f = pl.kernel(
        _sc_body,
        out_type=jax.ShapeDtypeStruct((NC, N, H1), jnp.float32),
        mesh=mesh,
        scratch_types=[
            pltpu.VMEM_SHARED((RN,), jnp.float32),
            pltpu.VMEM_SHARED((N, H1), jnp.float32),
            pltpu.VMEM((CG, GRP), jnp.int32),
            pltpu.VMEM((CG, GRP), jnp.int32),
            pltpu.VMEM((CE,), jnp.float32),
            pltpu.VMEM((CE, H1), jnp.float32),
            pltpu.VMEM((GRP,), jnp.float32),
            pltpu.VMEM((ZR, H1), jnp.float32),
            pltpu.VMEM((ZF,), jnp.float32),
            pltpu.VMEM((ZF,), jnp.float32),
            pltpu.SemaphoreType.DMA,
            pltpu.SemaphoreType.DMA,
            pltpu.SemaphoreType.DMA,
            pltpu.SemaphoreType.DMA,
        ],
        compiler_params=pltpu.CompilerParams(use_tc_tiling_on_sc=False),
    )
    return f(toh, s2d, wflat)


def _combine_body(parts_ref, bias_ref, out_ref):
    out_ref[...] = parts_ref[0] + parts_ref[1] + bias_ref[...]


BR = 2000


def _combine(parts, bias2):
    return pl.pallas_call(
        _combine_body,
        out_shape=jax.ShapeDtypeStruct((N, H1), jnp.float32),
        grid=(N // BR,),
        in_specs=[
            pl.BlockSpec((NC, BR, H1), lambda i: (0, i, 0)),
            pl.BlockSpec((1, H1), lambda i: (0, 0)),
        ],
        out_specs=pl.BlockSpec((BR, H1), lambda i: (i, 0)),
    )(parts, bias2)


def kernel(triples, weights, bias):
    toh = (triples[:, 1] * N + triples[:, 2]).reshape(NGRP, GRP)
    s2d = triples[:, 0].reshape(NGRP, GRP)
    wflat = weights.reshape(RN, H1)
    parts = _sc_call(toh, s2d, wflat)
    return _combine(parts, bias.reshape(1, H1))


# speedup vs baseline: 8.2348x; 1.0084x over previous
import functools

import jax
import jax.numpy as jnp
from jax import lax
from jax.experimental import pallas as pl
from jax.experimental.pallas import tpu as pltpu
from jax.experimental.pallas import tpu_sc as plsc

N = 50000
R = 16
H1 = 16
E = N * 32
RN = R * N

L = 16
NC = 2
NS = 16
NW = NC * NS

GRP = 128
CG = 10
CE = CG * GRP
NGRP = E // GRP
NCHUNK = NGRP // CG

CS_PER_TILE = RN // NS
ROWS_PER_TILE = N // NS
ZF = 2000
ZR = 125


def _sc_body(toh, s2d, wf, parts,
             colsum_sh, out_sh,
             frbuf, to2d, vals1, rows,
             ones1, zrow, zflat, zflat2, vsem, rsem, ssem, isem):
    cid = lax.axis_index("c")
    sid = lax.axis_index("s")
    wid = sid * NC + cid

    ones16 = jnp.ones((L,), jnp.float32)
    zeros16 = jnp.zeros((L,), jnp.float32)
    for i in range(GRP // L):
        ones1[pl.ds(i * L, L)] = ones16
    for r in range(ZR):
        zrow[r, :] = zeros16

    @pl.loop(0, ZF // L)
    def _zfill(i):
        zflat[pl.ds(i * L, L)] = jnp.zeros((L,), jnp.float32)

    @pl.loop(0, CS_PER_TILE // ZF)
    def _zero_cs(k):
        pltpu.sync_copy(zflat, colsum_sh.at[pl.ds(sid * CS_PER_TILE + k * ZF, ZF)])

    @pl.loop(0, ROWS_PER_TILE // ZR)
    def _zero_out(k):
        pltpu.sync_copy(zrow, out_sh.at[pl.ds(sid * ROWS_PER_TILE + k * ZR, ZR), :])

    plsc.subcore_barrier()

    @pl.loop(sid, NCHUNK, step=2 * NS)
    def _count(c0):
        pltpu.sync_copy(toh.at[pl.ds(c0 * CG, CG), :], to2d)
        d0 = [pltpu.async_copy(ones1, colsum_sh.at[to2d.at[j]], ssem, add=True)
              for j in range(CG)]
        c1 = c0 + NS

        @pl.when(c1 < NCHUNK)
        def _second():
            pltpu.sync_copy(toh.at[pl.ds(c1 * CG, CG), :], frbuf)
            d1 = [pltpu.async_copy(ones1, colsum_sh.at[frbuf.at[j]], vsem,
                                   add=True)
                  for j in range(CG)]
            for d in d1:
                d.wait()

        for d in d0:
            d.wait()

    plsc.subcore_barrier()

    NSL = CS_PER_TILE // ZF

    @pl.loop(0, NSL - 1, step=2)
    def _recip(k):
        sl0 = pl.ds(sid * CS_PER_TILE + k * ZF, ZF)
        sl1 = pl.ds(sid * CS_PER_TILE + (k + 1) * ZF, ZF)
        l0 = pltpu.async_copy(colsum_sh.at[sl0], zflat, vsem)
        l1 = pltpu.async_copy(colsum_sh.at[sl1], zflat2, rsem)
        l0.wait()

        @pl.loop(0, ZF // L)
        def _inv0(i):
            zflat[pl.ds(i * L, L)] = 1.0 / zflat[pl.ds(i * L, L)]

        s0 = pltpu.async_copy(zflat, colsum_sh.at[sl0], vsem)
        l1.wait()

        @pl.loop(0, ZF // L)
        def _inv1(i):
            zflat2[pl.ds(i * L, L)] = 1.0 / zflat2[pl.ds(i * L, L)]

        s1 = pltpu.async_copy(zflat2, colsum_sh.at[sl1], rsem)
        s0.wait()
        s1.wait()

    if NSL % 2 == 1:
        slt = pl.ds(sid * CS_PER_TILE + (NSL - 1) * ZF, ZF)
        pltpu.sync_copy(colsum_sh.at[slt], zflat)

        @pl.loop(0, ZF // L)
        def _invt(i):
            zflat[pl.ds(i * L, L)] = 1.0 / zflat[pl.ds(i * L, L)]

        pltpu.sync_copy(zflat, colsum_sh.at[slt])

    plsc.subcore_barrier()

    @pl.loop(wid, NCHUNK, step=NW)
    def _process(chunk):
        i0 = pltpu.async_copy(toh.at[pl.ds(chunk * CG, CG), :], to2d, isem)
        i1 = pltpu.async_copy(s2d.at[pl.ds(chunk * CG, CG), :], frbuf, isem)
        i0.wait()
        i1.wait()
        vdescs = [pltpu.async_copy(colsum_sh.at[to2d.at[j]],
                                   vals1.at[pl.ds(j * GRP, GRP)], vsem)
                  for j in range(CG)]
        rdescs = [pltpu.async_copy(wf.at[to2d.at[j]],
                                   rows.at[pl.ds(j * GRP, GRP), :], rsem)
                  for j in range(CG)]
        sdescs = []
        for j in range(CG):
            vdescs[j].wait()
            rdescs[j].wait()

            @pl.loop(0, GRP // L)
            def _scale(g):
                e0 = j * GRP + g * L
                v16 = vals1[pl.ds(e0, L)]
                for t in range(L):
                    rows[e0 + t, :] = rows[e0 + t, :] * v16[t]

            sdescs.append(
                pltpu.async_copy(rows.at[pl.ds(j * GRP, GRP), :],
                                 out_sh.at[frbuf.at[j]], ssem, add=True))
        for d in sdescs:
            d.wait()

    plsc.subcore_barrier()

    pltpu.sync_copy(out_sh.at[pl.ds(sid * ROWS_PER_TILE, ROWS_PER_TILE), :],
                    parts.at[cid, pl.ds(sid * ROWS_PER_TILE, ROWS_PER_TILE), :])


@functools.partial(jax.jit, static_argnames=())
def _sc_call(toh, s2d, wflat):
    mesh = plsc.VectorSubcoreMesh(core_axis_name="c", subcore_axis_name="s",
                                  num_cores=NC, num_subcores=NS)
    f = pl.kernel(
        _sc_body,
        out_type=jax.ShapeDtypeStruct((NC, N, H1), jnp.float32),
        mesh=mesh,
        scratch_types=[
            pltpu.VMEM_SHARED((RN,), jnp.float32),
            pltpu.VMEM_SHARED((N, H1), jnp.float32),
            pltpu.VMEM((CG, GRP), jnp.int32),
            pltpu.VMEM((CG, GRP), jnp.int32),
            pltpu.VMEM((CE,), jnp.float32),
            pltpu.VMEM((CE, H1), jnp.float32),
            pltpu.VMEM((GRP,), jnp.float32),
            pltpu.VMEM((ZR, H1), jnp.float32),
            pltpu.VMEM((ZF,), jnp.float32),
            pltpu.VMEM((ZF,), jnp.float32),
            pltpu.SemaphoreType.DMA,
            pltpu.SemaphoreType.DMA,
            pltpu.SemaphoreType.DMA,
            pltpu.SemaphoreType.DMA,
        ],
        compiler_params=pltpu.CompilerParams(use_tc_tiling_on_sc=False),
    )
    return f(toh, s2d, wflat)


def _combine_body(parts_ref, bias_ref, out_ref):
    out_ref[...] = parts_ref[0] + parts_ref[1] + bias_ref[...]


BR = 2000


def _combine(parts, bias2):
    return pl.pallas_call(
        _combine_body,
        out_shape=jax.ShapeDtypeStruct((N, H1), jnp.float32),
        grid=(N // BR,),
        in_specs=[
            pl.BlockSpec((NC, BR, H1), lambda i: (0, i, 0)),
            pl.BlockSpec((1, H1), lambda i: (0, 0)),
        ],
        out_specs=pl.BlockSpec((BR, H1), lambda i: (i, 0)),
    )(parts, bias2)


def kernel(triples, weights, bias):
    toh = (triples[:, 1] * N + triples[:, 2]).reshape(NGRP, GRP)
    s2d = triples[:, 0].reshape(NGRP, GRP)
    wflat = weights.reshape(RN, H1)
    parts = _sc_call(toh, s2d, wflat)
    return _combine(parts, bias.reshape(1, H1))
